# baseline (speedup 1.0000x reference)
import jax
import jax.numpy as jnp
from jax import lax
from jax.experimental import pallas as pl
from jax.experimental.pallas import tpu as pltpu
from jax.experimental.pallas import tpu_sc as plsc

EMBED_DIM = 64
PAIR_DIM = 2 * EMBED_DIM
BATCH = 16384
NUM_TABLES = 8
NC, NS = 2, 16
NW = NC * NS
B_PER_W = BATCH // NW
CHUNK = 128
NCHUNK = B_PER_W // CHUNK
NBUF = 3
NSTAGE = NUM_TABLES * NCHUNK


def _body(*refs):
    idx_refs = refs[:NUM_TABLES]
    table_refs = refs[NUM_TABLES:2 * NUM_TABLES]
    out_refs = refs[2 * NUM_TABLES:3 * NUM_TABLES]
    rest = refs[3 * NUM_TABLES:]
    idx_v = rest[0]
    rows = rest[1:1 + NBUF]
    sem_i = rest[1 + NBUF]
    sem_g = rest[2 + NBUF:2 + 2 * NBUF]
    sem_s = rest[2 + 2 * NBUF:2 + 3 * NBUF]

    wid = lax.axis_index("s") * NC + lax.axis_index("c")
    base = wid * B_PER_W

    for t in range(NUM_TABLES):
        pltpu.async_copy(idx_refs[t].at[wid], idx_v.at[t], sem_i)
    for t in range(NUM_TABLES):
        pltpu.make_async_copy(idx_refs[t].at[wid], idx_v.at[t], sem_i).wait()

    def gather_args(s):
        t, j = divmod(s, NCHUNK)
        b = s % NBUF
        return (table_refs[t].at[idx_v.at[t].at[j]], rows[b], sem_g[b])

    def store_args(s):
        t, j = divmod(s, NCHUNK)
        b = s % NBUF
        return (rows[b],
                out_refs[t].at[pl.ds(base + j * CHUNK, CHUNK)],
                sem_s[b])

    for s in range(NBUF):
        pltpu.async_copy(*gather_args(s))
    for s in range(NSTAGE):
        pltpu.make_async_copy(*gather_args(s)).wait()
        pltpu.async_copy(*store_args(s))
        nxt = s + NBUF
        if nxt < NSTAGE:
            pltpu.make_async_copy(*store_args(nxt - NBUF)).wait()
            pltpu.async_copy(*gather_args(nxt))
    for s in range(NSTAGE - NBUF, NSTAGE):
        pltpu.make_async_copy(*store_args(s)).wait()


@jax.jit
def _gather_all(*args):
    idxs = tuple((a >> 1).reshape(NW, NCHUNK, CHUNK)
                 for a in args[:NUM_TABLES])
    tables = tuple(t.reshape(t.shape[0] // 2, PAIR_DIM)
                   for t in args[NUM_TABLES:])
    mesh = plsc.VectorSubcoreMesh(
        core_axis_name="c", subcore_axis_name="s",
        num_cores=NC, num_subcores=NS)
    out_type = tuple(
        jax.ShapeDtypeStruct((BATCH, PAIR_DIM), jnp.float32)
        for _ in range(NUM_TABLES))
    scratch = [pltpu.VMEM((NUM_TABLES, NCHUNK, CHUNK), jnp.int32)]
    scratch += [pltpu.VMEM((CHUNK, PAIR_DIM), jnp.float32)
                for _ in range(NBUF)]
    scratch += [pltpu.SemaphoreType.DMA for _ in range(1 + 2 * NBUF)]
    return pl.kernel(
        _body,
        out_type=out_type,
        mesh=mesh,
        compiler_params=pltpu.CompilerParams(use_tc_tiling_on_sc=True),
        scratch_types=scratch,
    )(*idxs, *tables)


def kernel(pat_idx, vis_idx, symp_idx, proc_idx, dis_idx, med_idx, anat_idx,
           pharma_idx, pat_table, vis_table, symp_table, proc_table,
           dis_table, med_table, anat_table, pharma_table):
    idx_in = (pat_idx, vis_idx, symp_idx, proc_idx, dis_idx, med_idx,
              anat_idx, pharma_idx)
    pairs = _gather_all(
        *idx_in, pat_table, vis_table, symp_table, proc_table,
        dis_table, med_table, anat_table, pharma_table)
    outs = tuple(
        jnp.where((ix & 1)[:, None] == 1, p[:, EMBED_DIM:], p[:, :EMBED_DIM])
        for ix, p in zip(idx_in, pairs))
    x_pat, x_vis, x_symp, x_proc, x_dis, x_med, x_anat, x_pharma = outs
    return (x_pat, x_vis, x_proc, x_dis, x_med, x_symp, x_anat, x_pharma)


# speedup vs baseline: 1.0690x; 1.0690x over previous
import jax
import jax.numpy as jnp
from jax import lax
from jax.experimental import pallas as pl
from jax.experimental.pallas import tpu as pltpu
from jax.experimental.pallas import tpu_sc as plsc

EMBED_DIM = 64
BATCH = 16384
NC, NS = 2, 16
NW = NC * NS
B_PER_W = BATCH // NW
CHUNK = 128
NCHUNK = B_PER_W // CHUNK
NBUF = 3


def _body(idx_ref, table_ref, out_ref, idx_v, *rest):
    rows = rest[:NBUF]
    sem_i = rest[NBUF]
    sem_g = rest[NBUF + 1:2 * NBUF + 1]
    sem_s = rest[2 * NBUF + 1:]

    wid = lax.axis_index("s") * NC + lax.axis_index("c")
    base = wid * B_PER_W

    pltpu.async_copy(idx_ref.at[wid], idx_v, sem_i)
    pltpu.make_async_copy(idx_ref.at[wid], idx_v, sem_i).wait()

    def gather_args(j):
        b = j % NBUF
        return (table_ref.at[idx_v.at[j]], rows[b], sem_g[b])

    def store_args(j):
        b = j % NBUF
        return (rows[b], out_ref.at[pl.ds(base + j * CHUNK, CHUNK)], sem_s[b])

    for j in range(min(NBUF, NCHUNK)):
        pltpu.async_copy(*gather_args(j))
    for j in range(NCHUNK):
        pltpu.make_async_copy(*gather_args(j)).wait()
        pltpu.async_copy(*store_args(j))
        nxt = j + NBUF
        if nxt < NCHUNK:
            pltpu.make_async_copy(*store_args(nxt - NBUF)).wait()
            pltpu.async_copy(*gather_args(nxt))
    for j in range(max(0, NCHUNK - NBUF), NCHUNK):
        pltpu.make_async_copy(*store_args(j)).wait()


def _gather_one(idx, table):
    mesh = plsc.VectorSubcoreMesh(
        core_axis_name="c", subcore_axis_name="s",
        num_cores=NC, num_subcores=NS)
    scratch = [pltpu.VMEM((NCHUNK, CHUNK), jnp.int32)]
    scratch += [pltpu.VMEM((CHUNK, EMBED_DIM), jnp.float32)
                for _ in range(NBUF)]
    scratch += [pltpu.SemaphoreType.DMA for _ in range(1 + 2 * NBUF)]
    return pl.kernel(
        _body,
        out_type=jax.ShapeDtypeStruct((BATCH, EMBED_DIM), jnp.float32),
        mesh=mesh,
        compiler_params=pltpu.CompilerParams(use_tc_tiling_on_sc=False),
        scratch_types=scratch,
        name=f"sc_gather_v{table.shape[0]}",
    )(idx.reshape(NW, NCHUNK, CHUNK), table)


@jax.jit
def _gather_all(*args):
    idxs = args[:8]
    tables = args[8:]
    return tuple(_gather_one(i, t) for i, t in zip(idxs, tables))


def kernel(pat_idx, vis_idx, symp_idx, proc_idx, dis_idx, med_idx, anat_idx,
           pharma_idx, pat_table, vis_table, symp_table, proc_table,
           dis_table, med_table, anat_table, pharma_table):
    outs = _gather_all(
        pat_idx, vis_idx, symp_idx, proc_idx, dis_idx, med_idx, anat_idx,
        pharma_idx, pat_table, vis_table, symp_table, proc_table,
        dis_table, med_table, anat_table, pharma_table)
    x_pat, x_vis, x_symp, x_proc, x_dis, x_med, x_anat, x_pharma = outs
    return (x_pat, x_vis, x_proc, x_dis, x_med, x_symp, x_anat, x_pharma)


# speedup vs baseline: 1.8195x; 1.7021x over previous
import jax
import jax.numpy as jnp
from jax import lax
from jax.experimental import pallas as pl
from jax.experimental.pallas import tpu as pltpu
from jax.experimental.pallas import tpu_sc as plsc

EMBED_DIM = 64
BATCH = 16384
NC, NS = 2, 16
NW = NC * NS
B_PER_W = BATCH // NW
CHUNK = 128
NCHUNK = B_PER_W // CHUNK
NBUF = 3

V_VIS = 1000000
BLK = 512
NGRP = B_PER_W // 16
DUMP = BATCH


def _body_small(idx_ref, table_ref, out_ref, idx_v, *rest):
    rows = rest[:NBUF]
    sem_i = rest[NBUF]
    sem_g = rest[NBUF + 1:2 * NBUF + 1]
    sem_s = rest[2 * NBUF + 1:]

    wid = lax.axis_index("s") * NC + lax.axis_index("c")
    base = wid * B_PER_W

    pltpu.async_copy(idx_ref.at[wid], idx_v, sem_i)
    pltpu.make_async_copy(idx_ref.at[wid], idx_v, sem_i).wait()

    def gather_args(j):
        b = j % NBUF
        return (table_ref.at[idx_v.at[j]], rows[b], sem_g[b])

    def store_args(j):
        b = j % NBUF
        return (rows[b], out_ref.at[pl.ds(base + j * CHUNK, CHUNK)], sem_s[b])

    for j in range(min(NBUF, NCHUNK)):
        pltpu.async_copy(*gather_args(j))
    for j in range(NCHUNK):
        pltpu.make_async_copy(*gather_args(j)).wait()
        pltpu.async_copy(*store_args(j))
        nxt = j + NBUF
        if nxt < NCHUNK:
            pltpu.make_async_copy(*store_args(nxt - NBUF)).wait()
            pltpu.async_copy(*gather_args(nxt))
    for j in range(max(0, NCHUNK - NBUF), NCHUNK):
        pltpu.make_async_copy(*store_args(j)).wait()


def _gather_small(idx, table):
    mesh = plsc.VectorSubcoreMesh(
        core_axis_name="c", subcore_axis_name="s",
        num_cores=NC, num_subcores=NS)
    scratch = [pltpu.VMEM((NCHUNK, CHUNK), jnp.int32)]
    scratch += [pltpu.VMEM((CHUNK, EMBED_DIM), jnp.float32)
                for _ in range(NBUF)]
    scratch += [pltpu.SemaphoreType.DMA for _ in range(1 + 2 * NBUF)]
    return pl.kernel(
        _body_small,
        out_type=jax.ShapeDtypeStruct((BATCH, EMBED_DIM), jnp.float32),
        mesh=mesh,
        compiler_params=pltpu.CompilerParams(use_tc_tiling_on_sc=False),
        scratch_types=scratch,
        name=f"sc_gather_v{table.shape[0]}",
    )(idx.reshape(NW, NCHUNK, CHUNK), table)


TAIL = (V_VIS // CHUNK) * CHUNK


def _body_visit(vs_ref, bs_ref, tabT_ref, out_ref,
                vs_v, bs_v, ring, tail_v, stage0, stage1, bpos,
                sem_l, sem_r0, sem_r1, sem_s0, sem_s1):
    wid = lax.axis_index("s") * NC + lax.axis_index("c")

    pltpu.async_copy(vs_ref.at[wid], vs_v, sem_l)
    pltpu.async_copy(bs_ref.at[wid], bs_v, sem_l)
    pltpu.async_copy(tabT_ref.at[:, pl.ds(TAIL, V_VIS - TAIL)], tail_v, sem_l)
    pltpu.make_async_copy(vs_ref.at[wid], vs_v, sem_l).wait()
    pltpu.make_async_copy(bs_ref.at[wid], bs_v, sem_l).wait()
    pltpu.make_async_copy(
        tabT_ref.at[:, pl.ds(TAIL, V_VIS - TAIL)], tail_v, sem_l).wait()

    v_lo = jnp.minimum(vs_v[0, :][0], TAIL - 1)
    v_hi = jnp.minimum(vs_v[NGRP - 1, :][15], TAIL - 1)
    s0 = (v_lo // BLK) * BLK
    nblk = (v_hi - s0) // BLK + 1

    def blk_start(k):
        return pl.multiple_of(s0 + k * BLK, BLK)

    def issue_blk(k, slot):
        pltpu.async_copy(
            tabT_ref.at[:, pl.ds(blk_start(k), BLK)],
            ring.at[:, pl.ds(slot * BLK, BLK)],
            sem_r0 if slot == 0 else sem_r1)

    def wait_blk(k, slot):
        pltpu.make_async_copy(
            tabT_ref.at[:, pl.ds(blk_start(k), BLK)],
            ring.at[:, pl.ds(slot * BLK, BLK)],
            sem_r0 if slot == 0 else sem_r1).wait()

    issue_blk(0, 0)

    dpat = [jax.lax.iota(jnp.int32, 16) + 16 * kk for kk in range(4)]
    iota16 = jax.lax.iota(jnp.int32, 16)

    def scatter_descr(which):
        st = stage0 if which == 0 else stage1
        sem = sem_s0 if which == 0 else sem_s1
        return (st, out_ref.at[bpos.at[which]], sem)

    def emit_event(src, colv, pos, args):
        e2, c02, c12 = args
        use0 = (e2 % 2) == 0

        @pl.when(jnp.logical_and(e2 >= 2, use0))
        def _():
            pltpu.make_async_copy(*scatter_descr(0)).wait()

        @pl.when(jnp.logical_and(e2 >= 2, jnp.logical_not(use0)))
        def _():
            pltpu.make_async_copy(*scatter_descr(1)).wait()

        def fill(st):
            for i in range(16):
                cvec = jnp.full((16,), 1, jnp.int32) * colv[i]
                for kk in range(4):
                    vals = plsc.load_gather(src, [dpat[kk], cvec])
                    st[i, pl.ds(16 * kk, 16)] = vals

        @pl.when(use0)
        def _():
            fill(stage0)
            bpos[0, :] = pos
            pltpu.async_copy(*scatter_descr(0))

        @pl.when(jnp.logical_not(use0))
        def _():
            fill(stage1)
            bpos[1, :] = pos
            pltpu.async_copy(*scatter_descr(1))

        return (e2 + 1,
                c02 + jnp.where(use0, 1, 0),
                c12 + jnp.where(use0, 0, 1))

    def outer(k, carry):
        slot_next_is0 = ((k + 1) % 2) == 0

        @pl.when(jnp.logical_and(k + 1 < nblk, slot_next_is0))
        def _():
            issue_blk(k + 1, 0)

        @pl.when(jnp.logical_and(k + 1 < nblk, jnp.logical_not(slot_next_is0)))
        def _():
            issue_blk(k + 1, 1)

        @pl.when((k % 2) == 0)
        def _():
            wait_blk(k, 0)

        @pl.when((k % 2) != 0)
        def _():
            wait_blk(k, 1)

        start = blk_start(k)
        base_col = (k % 2) * BLK - start

        def group(g, carry_in):
            grow = vs_v[g, :]
            overlap = jnp.logical_and(grow[0] < start + BLK,
                                      grow[15] >= start)

            def do(args):
                v16 = vs_v[g, :]
                b16 = bs_v[g, :]
                mask = jnp.logical_and(v16 >= start, v16 < start + BLK)
                pos = jnp.where(mask, b16, DUMP + iota16)
                colv = jnp.clip(v16 + base_col, 0, 2 * BLK - 1)
                return emit_event(ring, colv, pos, args)

            return lax.cond(overlap, do, lambda a: a, carry_in)

        return lax.fori_loop(0, NGRP, group, carry)

    carry = lax.fori_loop(0, nblk, outer, (0, 0, 0))

    def tail_group(g, carry_in):
        grow = vs_v[g, :]

        def do(args):
            v16 = vs_v[g, :]
            b16 = bs_v[g, :]
            mask = v16 >= TAIL
            pos = jnp.where(mask, b16, DUMP + iota16)
            colv = jnp.clip(v16 - TAIL, 0, V_VIS - TAIL - 1)
            return emit_event(tail_v, colv, pos, args)

        return lax.cond(grow[15] >= TAIL, do, lambda a: a, carry_in)

    e, c0, c1 = lax.fori_loop(0, NGRP, tail_group, carry)

    def drain0(_, carry):
        pltpu.make_async_copy(*scatter_descr(0)).wait()
        return carry

    def drain1(_, carry):
        pltpu.make_async_copy(*scatter_descr(1)).wait()
        return carry

    lax.fori_loop(0, jnp.minimum(c0, 1), drain0, 0)
    lax.fori_loop(0, jnp.minimum(c1, 1), drain1, 0)


def _gather_visit(vis_idx, vis_table):
    v_s, b_s = lax.sort_key_val(vis_idx,
                                jnp.arange(BATCH, dtype=jnp.int32))
    mesh = plsc.VectorSubcoreMesh(
        core_axis_name="c", subcore_axis_name="s",
        num_cores=NC, num_subcores=NS)
    out = pl.kernel(
        _body_visit,
        out_type=jax.ShapeDtypeStruct((BATCH + 16, 2 * EMBED_DIM),
                                      jnp.float32),
        mesh=mesh,
        compiler_params=pltpu.CompilerParams(
            use_tc_tiling_on_sc=True, needs_layout_passes=False),
        scratch_types=[
            pltpu.VMEM((NGRP, 16), jnp.int32),
            pltpu.VMEM((NGRP, 16), jnp.int32),
            pltpu.VMEM((EMBED_DIM, 2 * BLK), jnp.float32),
            pltpu.VMEM((EMBED_DIM, V_VIS - TAIL), jnp.float32),
            pltpu.VMEM((16, 2 * EMBED_DIM), jnp.float32),
            pltpu.VMEM((16, 2 * EMBED_DIM), jnp.float32),
            pltpu.VMEM((2, 16), jnp.int32),
            pltpu.SemaphoreType.DMA,
            pltpu.SemaphoreType.DMA,
            pltpu.SemaphoreType.DMA,
            pltpu.SemaphoreType.DMA,
            pltpu.SemaphoreType.DMA,
        ],
        name="sc_stream_visit",
    )(v_s.reshape(NW, NGRP, 16), b_s.reshape(NW, NGRP, 16), vis_table.T)
    return out[:BATCH, :EMBED_DIM]


@jax.jit
def _gather_all(*args):
    idxs = args[:8]
    tables = args[8:]
    outs = []
    for i, (ix, t) in enumerate(zip(idxs, tables)):
        if i == 1:
            outs.append(_gather_visit(ix, t))
        else:
            outs.append(_gather_small(ix, t))
    return tuple(outs)


def kernel(pat_idx, vis_idx, symp_idx, proc_idx, dis_idx, med_idx, anat_idx,
           pharma_idx, pat_table, vis_table, symp_table, proc_table,
           dis_table, med_table, anat_table, pharma_table):
    outs = _gather_all(
        pat_idx, vis_idx, symp_idx, proc_idx, dis_idx, med_idx, anat_idx,
        pharma_idx, pat_table, vis_table, symp_table, proc_table,
        dis_table, med_table, anat_table, pharma_table)
    x_pat, x_vis, x_symp, x_proc, x_dis, x_med, x_anat, x_pharma = outs
    return (x_pat, x_vis, x_proc, x_dis, x_med, x_symp, x_anat, x_pharma)
